# baseline (speedup 1.0000x reference)
import functools

import jax
import jax.numpy as jnp
from jax import lax
from jax.experimental import pallas as pl
from jax.experimental.pallas import tpu as pltpu
from jax.experimental.pallas import tpu_sc as plsc

B = 16384
D = 64


def _sc_gather_body(user_hbm, item_hbm, ut_hbm, it_hbm, uout_hbm, iout_hbm,
                    idx_u, idx_i, rows_u, rows_i, sem_u, sem_i, nc, bpw):
    wid = lax.axis_index("s") * nc + lax.axis_index("c")
    base = wid * bpw
    pltpu.sync_copy(user_hbm.at[pl.ds(base, bpw)], idx_u)
    pltpu.sync_copy(item_hbm.at[pl.ds(base, bpw)], idx_i)
    cu = pltpu.async_copy(ut_hbm.at[idx_u], rows_u, sem_u)
    ci = pltpu.async_copy(it_hbm.at[idx_i], rows_i, sem_i)
    cu.wait()
    ci.wait()
    pltpu.sync_copy(rows_u, uout_hbm.at[pl.ds(base, bpw)])
    pltpu.sync_copy(rows_i, iout_hbm.at[pl.ds(base, bpw)])


@jax.jit
def _sc_gather(user, item, user_table, item_table):
    info = plsc.get_sparse_core_info()
    nc, ns = info.num_cores, info.num_subcores
    nw = nc * ns
    bpw = B // nw
    mesh = plsc.VectorSubcoreMesh(core_axis_name="c", subcore_axis_name="s")
    body = functools.partial(_sc_gather_body, nc=nc, bpw=bpw)
    k = pl.kernel(
        body,
        out_type=[
            jax.ShapeDtypeStruct((B, D), jnp.float32),
            jax.ShapeDtypeStruct((B, D), jnp.float32),
        ],
        mesh=mesh,
        compiler_params=pltpu.CompilerParams(use_tc_tiling_on_sc=False),
        scratch_types=[
            pltpu.VMEM((bpw,), jnp.int32),
            pltpu.VMEM((bpw,), jnp.int32),
            pltpu.VMEM((bpw, D), jnp.float32),
            pltpu.VMEM((bpw, D), jnp.float32),
            pltpu.SemaphoreType.DMA,
            pltpu.SemaphoreType.DMA,
        ],
    )
    return k(user, item, user_table, item_table)


def _mlp_body(u_ref, i_ref, w1a_ref, w1b_ref, b1_ref, w2_ref, b2_ref,
              w3_ref, b3_ref, out_ref):
    u = u_ref[...]
    i = i_ref[...]
    h = u @ w1a_ref[...] + i @ w1b_ref[...] + b1_ref[...]
    h = jnp.maximum(h, 0.0)
    h = jnp.maximum(h @ w2_ref[...] + b2_ref[...], 0.0)
    out_ref[...] = h @ w3_ref[...] + b3_ref[...]


@jax.jit
def _mlp(u, i, W1, b1, W2, b2, W3, b3):
    blk = 4096
    grid = B // blk
    w1a = W1[:D]
    w1b = W1[D:]
    full = lambda s: pl.BlockSpec(s, lambda j: (0, 0))
    out = pl.pallas_call(
        _mlp_body,
        grid=(grid,),
        in_specs=[
            pl.BlockSpec((blk, D), lambda j: (j, 0)),
            pl.BlockSpec((blk, D), lambda j: (j, 0)),
            full((D, 64)),
            full((D, 64)),
            full((1, 64)),
            full((64, 32)),
            full((1, 32)),
            full((32, 1)),
            full((1, 1)),
        ],
        out_specs=pl.BlockSpec((blk, 1), lambda j: (j, 0)),
        out_shape=jax.ShapeDtypeStruct((B, 1), jnp.float32),
    )(u, i, w1a, w1b, b1.reshape(1, 64), W2, b2.reshape(1, 32), W3,
      b3.reshape(1, 1))
    return out


def kernel(user, item, user_table, item_table, W1, b1, W2, b2, W3, b3):
    user = user.astype(jnp.int32)
    item = item.astype(jnp.int32)
    u, i = _sc_gather(user, item, user_table, item_table)
    out = _mlp(u, i, W1, b1, W2, b2, W3, b3)
    return jnp.squeeze(out, axis=-1)


# speedup vs baseline: 1.6249x; 1.6249x over previous
import functools

import jax
import jax.numpy as jnp
from jax import lax
from jax.experimental import pallas as pl
from jax.experimental.pallas import tpu as pltpu
from jax.experimental.pallas import tpu_sc as plsc

B = 16384
D = 64
K = 32


CH = 256


def _fire(table_hbm, rows_v, idx_v, sem, jbase, dbase):
    for v in range(K // 16):
        rvec = idx_v[pl.ds(jbase + v * 16, 16)]
        for jj in range(16):
            r = rvec[jj]
            d = dbase + v * 16 + jj
            pltpu.async_copy(
                table_hbm.at[pl.ds(r, 1), :], rows_v.at[pl.ds(d, 1), :], sem)


def _drain(table_hbm, rows_v, sem, dbase):
    for jj in range(K):
        d = dbase + jj
        pltpu.make_async_copy(
            table_hbm.at[pl.ds(0, 1), :], rows_v.at[pl.ds(d, 1), :],
            sem).wait()


def _sc_gather_body(user_hbm, item_hbm, ut_hbm, it_hbm, uout_hbm, iout_hbm,
                    idx_u, idx_i, rows_u, rows_i, sem_u, sem_i, nc, bpw):
    wid = lax.axis_index("s") * nc + lax.axis_index("c")
    base = wid * bpw
    pltpu.sync_copy(user_hbm.at[pl.ds(base, bpw)], idx_u)
    pltpu.sync_copy(item_hbm.at[pl.ds(base, bpw)], idx_i)
    nb = CH // K
    for c in range(bpw // CH):
        cb = c * CH
        _fire(ut_hbm, rows_u, idx_u, sem_u, cb, 0)
        _fire(it_hbm, rows_i, idx_i, sem_i, cb, 0)

        def step(b, _, cb=cb):
            jb = cb + b * K
            db = b * K
            _fire(ut_hbm, rows_u, idx_u, sem_u, jb, db)
            _fire(it_hbm, rows_i, idx_i, sem_i, jb, db)
            _drain(ut_hbm, rows_u, sem_u, db - K)
            _drain(it_hbm, rows_i, sem_i, db - K)
            return 0

        lax.fori_loop(1, nb, step, 0)
        _drain(ut_hbm, rows_u, sem_u, CH - K)
        _drain(it_hbm, rows_i, sem_i, CH - K)
        pltpu.sync_copy(rows_u, uout_hbm.at[pl.ds(base + cb, CH)])
        pltpu.sync_copy(rows_i, iout_hbm.at[pl.ds(base + cb, CH)])


@jax.jit
def _sc_gather(user, item, user_table, item_table):
    info = plsc.get_sparse_core_info()
    nc, ns = info.num_cores, info.num_subcores
    nw = nc * ns
    bpw = B // nw
    mesh = plsc.VectorSubcoreMesh(core_axis_name="c", subcore_axis_name="s")
    body = functools.partial(_sc_gather_body, nc=nc, bpw=bpw)
    k = pl.kernel(
        body,
        out_type=[
            jax.ShapeDtypeStruct((B, D), jnp.float32),
            jax.ShapeDtypeStruct((B, D), jnp.float32),
        ],
        mesh=mesh,
        compiler_params=pltpu.CompilerParams(use_tc_tiling_on_sc=True),
        scratch_types=[
            pltpu.VMEM((bpw,), jnp.int32),
            pltpu.VMEM((bpw,), jnp.int32),
            pltpu.VMEM((CH, D), jnp.float32),
            pltpu.VMEM((CH, D), jnp.float32),
            pltpu.SemaphoreType.DMA,
            pltpu.SemaphoreType.DMA,
        ],
    )
    return k(user, item, user_table, item_table)


def _mlp_body(u_ref, i_ref, w1a_ref, w1b_ref, b1_ref, w2_ref, b2_ref,
              w3_ref, b3_ref, out_ref):
    u = u_ref[...]
    i = i_ref[...]
    h = u @ w1a_ref[...] + i @ w1b_ref[...] + b1_ref[...]
    h = jnp.maximum(h, 0.0)
    h = jnp.maximum(h @ w2_ref[...] + b2_ref[...], 0.0)
    out_ref[...] = h @ w3_ref[...] + b3_ref[...]


@jax.jit
def _mlp(u, i, W1, b1, W2, b2, W3, b3):
    blk = 4096
    grid = B // blk
    w1a = W1[:D]
    w1b = W1[D:]
    full = lambda s: pl.BlockSpec(s, lambda j: (0, 0))
    out = pl.pallas_call(
        _mlp_body,
        grid=(grid,),
        in_specs=[
            pl.BlockSpec((blk, D), lambda j: (j, 0)),
            pl.BlockSpec((blk, D), lambda j: (j, 0)),
            full((D, 64)),
            full((D, 64)),
            full((1, 64)),
            full((64, 32)),
            full((1, 32)),
            full((32, 1)),
            full((1, 1)),
        ],
        out_specs=pl.BlockSpec((blk, 1), lambda j: (j, 0)),
        out_shape=jax.ShapeDtypeStruct((B, 1), jnp.float32),
    )(u, i, w1a, w1b, b1.reshape(1, 64), W2, b2.reshape(1, 32), W3,
      b3.reshape(1, 1))
    return out


def kernel(user, item, user_table, item_table, W1, b1, W2, b2, W3, b3):
    user = user.astype(jnp.int32)
    item = item.astype(jnp.int32)
    u, i = _sc_gather(user, item, user_table, item_table)
    out = _mlp(u, i, W1, b1, W2, b2, W3, b3)
    return jnp.squeeze(out, axis=-1)
